# Initial kernel scaffold; baseline (speedup 1.0000x reference)
import jax
import jax.numpy as jnp
from jax.experimental import pallas as pl


def kernel(data, batch_sizes):
    raise NotImplementedError("write your pallas kernel here")


# speedup vs baseline: 2.5747x; 2.5747x over previous
import jax
import jax.numpy as jnp
from jax import lax
from jax.experimental import pallas as pl
from jax.experimental.pallas import tpu as pltpu
from jax.experimental.pallas import tpu_sc as plsc

BATCH = 16
MAX_LEN = 2048
D = 512
L = 16
NC, NS = 2, 16
NW = NC * NS
T_PER_W = MAX_LEN // NW
CHUNK_T = 4
CHUNK_ROWS = CHUNK_T * BATCH
NCHUNK = T_PER_W // CHUNK_T


def _body(data_hbm, bs_hbm, out_hbm, bs_v, idx_v, buf0, buf1, gsem, ssem):
    zero_row = data_hbm.shape[0] - 1
    wid = lax.axis_index("s") * NC + lax.axis_index("c")
    t0 = wid * T_PER_W

    pltpu.sync_copy(bs_hbm, bs_v)

    nvec = t0 // L

    def acc_body(j, a):
        return a + bs_v[pl.ds(j * L, L)]

    acc = lax.fori_loop(0, nvec, acc_body, jnp.zeros((L,), jnp.int32))
    off = jnp.sum(acc)

    iota = lax.iota(jnp.int32, L)
    for g in range(T_PER_W // L):
        bvec = bs_v[pl.ds(t0 + g * L, L)]
        cum = plsc.cumsum(bvec)
        excl = cum - bvec
        for k in range(L):
            b = bvec[k]
            base = off + excl[k]
            t_local = g * L + k
            idx_v[t_local // CHUNK_T, pl.ds((t_local % CHUNK_T) * L, L)] = (
                jnp.where(iota < b, base + iota, zero_row))
        off = off + cum[L - 1]

    bufs = (buf0, buf1)
    gh = [None] * NCHUNK
    sh = [None] * NCHUNK
    out0 = wid * (T_PER_W * BATCH)
    gh[0] = pltpu.async_copy(data_hbm.at[idx_v.at[0]], buf0, gsem)
    for c in range(NCHUNK):
        cur = bufs[c % 2]
        gh[c].wait()
        sh[c] = pltpu.async_copy(
            cur, out_hbm.at[pl.ds(out0 + c * CHUNK_ROWS, CHUNK_ROWS)], ssem)
        if c + 1 < NCHUNK:
            nxt = bufs[(c + 1) % 2]
            if c >= 1:
                sh[c - 1].wait()
            gh[c + 1] = pltpu.async_copy(data_hbm.at[idx_v.at[c + 1]], nxt, gsem)
    sh[NCHUNK - 1].wait()


def kernel(data, batch_sizes):
    total = data.shape[0]
    data_p = jnp.concatenate(
        [data, jnp.zeros((1, D), dtype=data.dtype)], axis=0)
    bs32 = batch_sizes.astype(jnp.int32)

    mesh = plsc.VectorSubcoreMesh(
        core_axis_name="c", subcore_axis_name="s", num_cores=NC,
        num_subcores=NS)
    out_flat = pl.kernel(
        _body,
        out_type=jax.ShapeDtypeStruct((MAX_LEN * BATCH, D), jnp.float32),
        mesh=mesh,
        compiler_params=pltpu.CompilerParams(needs_layout_passes=False),
        scratch_types=[
            pltpu.VMEM((MAX_LEN,), jnp.int32),
            pltpu.VMEM((NCHUNK, CHUNK_ROWS), jnp.int32),
            pltpu.VMEM((CHUNK_ROWS, D), jnp.float32),
            pltpu.VMEM((CHUNK_ROWS, D), jnp.float32),
            pltpu.SemaphoreType.DMA,
            pltpu.SemaphoreType.DMA,
        ],
    )(data_p, bs32)
    return out_flat.reshape(MAX_LEN, BATCH, D)


# speedup vs baseline: 2.5780x; 1.0013x over previous
import jax
import jax.numpy as jnp
from jax import lax
from jax.experimental import pallas as pl
from jax.experimental.pallas import tpu as pltpu
from jax.experimental.pallas import tpu_sc as plsc

BATCH = 16
MAX_LEN = 2048
D = 512
L = 16
NC, NS = 2, 16
NW = NC * NS
T_PER_W = MAX_LEN // NW
CHUNK_T = 4
CHUNK_ROWS = CHUNK_T * BATCH
NCHUNK = T_PER_W // CHUNK_T


def _body(data_hbm, bs_hbm, out_hbm, bs_v, idx_v, buf0, buf1, gsem, ssem):
    zero_row = data_hbm.shape[0] - 1
    wid = lax.axis_index("s") * NC + lax.axis_index("c")
    t0 = wid * T_PER_W

    pltpu.sync_copy(bs_hbm, bs_v)

    nvec = t0 // L

    def acc_body(j, a):
        return a + bs_v[pl.ds(j * L, L)]

    acc = lax.fori_loop(0, nvec, acc_body, jnp.zeros((L,), jnp.int32))
    off = jnp.sum(acc)

    iota = lax.iota(jnp.int32, L)
    for g in range(T_PER_W // L):
        bvec = bs_v[pl.ds(t0 + g * L, L)]
        cum = plsc.cumsum(bvec)
        excl = cum - bvec
        for k in range(L):
            b = bvec[k]
            base = off + excl[k]
            t_local = g * L + k
            idx_v[t_local, pl.ds(0, L)] = (
                jnp.where(iota < b, base + iota, zero_row))
        off = off + cum[L - 1]

    bufs = (buf0, buf1)
    gh = [None] * NCHUNK
    sh = [None] * NCHUNK
    out0 = wid * (T_PER_W * BATCH)

    def _gather(c, dst):
        return [pltpu.async_copy(
                    data_hbm.at[idx_v.at[c * CHUNK_T + s]],
                    dst.at[pl.ds(s * BATCH, BATCH)], gsem)
                for s in range(CHUNK_T)]

    gh[0] = _gather(0, buf0)
    for c in range(NCHUNK):
        cur = bufs[c % 2]
        for h in gh[c]:
            h.wait()
        sh[c] = pltpu.async_copy(
            cur, out_hbm.at[pl.ds(out0 + c * CHUNK_ROWS, CHUNK_ROWS)], ssem)
        if c + 1 < NCHUNK:
            nxt = bufs[(c + 1) % 2]
            if c >= 1:
                sh[c - 1].wait()
            gh[c + 1] = _gather(c + 1, nxt)
    sh[NCHUNK - 1].wait()


def kernel(data, batch_sizes):
    total = data.shape[0]
    data_p = jnp.concatenate(
        [data, jnp.zeros((1, D), dtype=data.dtype)], axis=0)
    bs32 = batch_sizes.astype(jnp.int32)

    mesh = plsc.VectorSubcoreMesh(
        core_axis_name="c", subcore_axis_name="s", num_cores=NC,
        num_subcores=NS)
    out_flat = pl.kernel(
        _body,
        out_type=jax.ShapeDtypeStruct((MAX_LEN * BATCH, D), jnp.float32),
        mesh=mesh,
        compiler_params=pltpu.CompilerParams(needs_layout_passes=False),
        scratch_types=[
            pltpu.VMEM((MAX_LEN,), jnp.int32),
            pltpu.VMEM((T_PER_W, L), jnp.int32),
            pltpu.VMEM((CHUNK_ROWS, D), jnp.float32),
            pltpu.VMEM((CHUNK_ROWS, D), jnp.float32),
            pltpu.SemaphoreType.DMA,
            pltpu.SemaphoreType.DMA,
        ],
    )(data_p, bs32)
    return out_flat.reshape(MAX_LEN, BATCH, D)


# speedup vs baseline: 11.7058x; 4.5406x over previous
import jax
import jax.numpy as jnp
from jax import lax
from jax.experimental import pallas as pl
from jax.experimental.pallas import tpu as pltpu
from jax.experimental.pallas import tpu_sc as plsc

BATCH = 16
MAX_LEN = 2048
D = 512
L = 16
NC, NS = 2, 16
NW = NC * NS
T_PER_W = MAX_LEN // NW
CHUNK_T = 4
CHUNK_ROWS = CHUNK_T * BATCH
NCHUNK = T_PER_W // CHUNK_T
PIECES = (16, 8, 4, 2, 1)


def _body(data_hbm, bs_hbm, zblk_hbm, out_hbm, bs_v, buf0, buf1,
          gsem0, gsem1, ssem0, ssem1):
    wid = lax.axis_index("s") * NC + lax.axis_index("c")
    t0 = wid * T_PER_W
    out0 = wid * (T_PER_W * BATCH)

    pltpu.sync_copy(bs_hbm, bs_v.at[pl.ds(0, MAX_LEN)])

    pltpu.sync_copy(zblk_hbm, buf0)
    pltpu.sync_copy(zblk_hbm, buf1)

    def acc_body(j, a):
        return a + bs_v[pl.ds(j * L, L)]

    acc = lax.fori_loop(0, t0 // L + T_PER_W // L, acc_body,
                        jnp.zeros((L,), jnp.int32))
    off_end = jnp.sum(acc)

    bufs = (buf0, buf1)
    gsems = (gsem0, gsem1)

    def gather_t(b, src0, buf, slot):
        for p in PIECES:
            pos = b & (~(2 * p - 1) & 31)

            @pl.when((b & p) != 0)
            def _(p=p, pos=pos):
                pltpu.async_copy(
                    data_hbm.at[pl.ds((src0 + pos) * D, p * D)],
                    buf.at[pl.ds((slot + pos) * D, p * D)],
                    gsems[_par[0]])

    def drain_t(b, buf, slot):
        for p in PIECES:
            pos = b & (~(2 * p - 1) & 31)

            @pl.when((b & p) != 0)
            def _(p=p, pos=pos):
                pltpu.make_async_copy(
                    data_hbm.at[pl.ds(0, p * D)],
                    buf.at[pl.ds((slot + pos) * D, p * D)],
                    gsems[_par[0]]).wait()

    ssems = (ssem0, ssem1)
    _par = [0]

    def drain_scatter(par):
        pltpu.make_async_copy(
            bufs[par], out_hbm.at[pl.ds(out0 * D, CHUNK_ROWS * D)],
            ssems[par]).wait()

    def do_chunk(c, par, off_after, first):
        _par[0] = par
        buf = bufs[par]

        @pl.when(jnp.logical_not(first))
        def _():
            drain_scatter(par)

        bvec = bs_v[pl.ds(t0 + c * CHUNK_T, L)]
        bs_k = [bvec[k] for k in range(CHUNK_T)]
        offs = [None] * CHUNK_T
        off = off_after
        for k in range(CHUNK_T - 1, -1, -1):
            off = off - bs_k[k]
            offs[k] = off
        for k in range(CHUNK_T):
            gather_t(bs_k[k], offs[k], buf, k * BATCH)
        for k in range(CHUNK_T):
            drain_t(bs_k[k], buf, k * BATCH)
        pltpu.async_copy(
            buf,
            out_hbm.at[pl.ds((out0 + c * CHUNK_ROWS) * D, CHUNK_ROWS * D)],
            ssems[par])
        return off

    def outer(i, off):
        c1 = (NCHUNK - 1) - 2 * i
        first = i == 0
        off = do_chunk(c1, 1, off, first)
        off = do_chunk(c1 - 1, 0, off, first)
        return off

    lax.fori_loop(0, NCHUNK // 2, outer, off_end)
    drain_scatter(0)
    drain_scatter(1)


def kernel(data, batch_sizes):
    bs32 = batch_sizes.astype(jnp.int32)
    zblk = jnp.zeros((CHUNK_ROWS * D,), jnp.float32)

    mesh = plsc.VectorSubcoreMesh(
        core_axis_name="c", subcore_axis_name="s", num_cores=NC,
        num_subcores=NS)
    out_flat = pl.kernel(
        _body,
        out_type=jax.ShapeDtypeStruct((MAX_LEN * BATCH * D,), jnp.float32),
        mesh=mesh,
        compiler_params=pltpu.CompilerParams(needs_layout_passes=False),
        scratch_types=[
            pltpu.VMEM((MAX_LEN + L,), jnp.int32),
            pltpu.VMEM((CHUNK_ROWS * D,), jnp.float32),
            pltpu.VMEM((CHUNK_ROWS * D,), jnp.float32),
            pltpu.SemaphoreType.DMA,
            pltpu.SemaphoreType.DMA,
            pltpu.SemaphoreType.DMA,
            pltpu.SemaphoreType.DMA,
        ],
    )(data.reshape(-1), bs32, zblk)
    return out_flat.reshape(MAX_LEN, BATCH, D)
